# baseline (speedup 1.0000x reference)
import dataclasses
import functools

import jax
import jax.numpy as jnp
from jax import lax
from jax.experimental import pallas as pl
from jax.experimental.pallas import tpu as pltpu
from jax.experimental.pallas import tpu_sc as plsc

_F = 768
_E = 256
_H = 512
_BAG = 32
_CHUNK = 4096
_NW = 32
_SPW = _CHUNK // _NW
_GRP = 32
_NGRP = _SPW // _GRP
_BLK = 1024


def _sc_hist_body(chunk_row0, idx_hbm, cnt_hbm, idx_v, cnt_a, cnt_b,
                  sem_a, sem_b):
    wid = lax.axis_index("s") * 2 + lax.axis_index("c")
    row0 = wid * _SPW
    pltpu.sync_copy(idx_hbm.at[pl.ds(chunk_row0 + row0, _SPW)], idx_v)

    def _dense_zero(cnt):
        @pl.loop(0, _GRP)
        def _zero_r(r):
            @pl.loop(0, _F, step=64)
            def _zero_c(i):
                z = jnp.zeros((16,), jnp.float32)
                for k in range(4):
                    cnt[r, pl.ds(i + 16 * k, 16)] = z

    ones = jnp.full((16,), 1.0, jnp.float32)
    zeros = jnp.zeros((16,), jnp.float32)
    bufs = [(cnt_a, sem_a), (cnt_b, sem_b)]
    copies = [None, None]

    _dense_zero(cnt_a)
    for g in range(_NGRP):
        cnt, sem = bufs[g % 2]
        if copies[g % 2] is not None:
            copies[g % 2].wait()
            pg = g - 2

            def _rezero(t, pg=pg, cnt=cnt):
                rows = jnp.full((16,), t, jnp.int32)
                a = idx_v[pg * _GRP + t, pl.ds(0, 16)]
                b = idx_v[pg * _GRP + t, pl.ds(16, 16)]
                plsc.store_scatter(cnt, [rows, a], zeros)
                plsc.store_scatter(cnt, [rows, b], zeros)

            pl.loop(0, _GRP)(_rezero)

        def _hist(t, g=g, cnt=cnt):
            for u in range(2):
                s = 2 * t + u
                rows = jnp.full((16,), s, jnp.int32)
                a = idx_v[g * _GRP + s, pl.ds(0, 16)]
                b = idx_v[g * _GRP + s, pl.ds(16, 16)]
                plsc.addupdate_scatter(cnt, [rows, a], ones)
                plsc.addupdate_scatter(cnt, [rows, b], ones)

        pl.loop(0, _GRP // 2)(_hist)
        copies[g % 2] = pltpu.async_copy(
            cnt, cnt_hbm.at[pl.ds(row0 + g * _GRP, _GRP)], sem)
        if g == 0:
            _dense_zero(cnt_b)

    copies[(_NGRP - 2) % 2].wait()
    copies[(_NGRP - 1) % 2].wait()


@functools.lru_cache(maxsize=None)
def _sc_hist(chunk_row0):
    cp = pltpu.CompilerParams()
    if "needs_layout_passes" in pltpu.CompilerParams.__dataclass_fields__:
        cp = dataclasses.replace(cp, needs_layout_passes=False)
    cp = dataclasses.replace(cp, use_tc_tiling_on_sc=True)
    return pl.kernel(
        functools.partial(_sc_hist_body, chunk_row0),
        compiler_params=cp,
        out_type=jax.ShapeDtypeStruct((_CHUNK, _F), jnp.float32),
        mesh=plsc.VectorSubcoreMesh(core_axis_name="c", subcore_axis_name="s"),
        scratch_types=[
            pltpu.VMEM((_SPW, _BAG), jnp.int32),
            pltpu.VMEM((_GRP, _F), jnp.float32),
            pltpu.VMEM((_GRP, _F), jnp.float32),
            pltpu.SemaphoreType.DMA,
            pltpu.SemaphoreType.DMA,
        ],
    )


def _mlp_body(cnt_ref, emb_ref, w1_ref, b1_ref, w2_ref, b2_ref, w3_ref,
              b3_ref, prev_ref, out_ref):
    del prev_ref
    c = cnt_ref[...].astype(jnp.bfloat16)
    x = jnp.dot(c, emb_ref[...], preferred_element_type=jnp.float32)
    h1 = jnp.dot(x.astype(jnp.bfloat16), w1_ref[...],
                 preferred_element_type=jnp.float32) + b1_ref[...]
    h1 = jnp.maximum(h1, 0.0).astype(jnp.bfloat16)
    h2 = jnp.dot(h1, w2_ref[...], preferred_element_type=jnp.float32) \
        + b2_ref[...]
    h2 = jnp.maximum(h2, 0.0)
    out_ref[...] = jnp.sum(h2 * w3_ref[...], axis=1, keepdims=True) \
        + b3_ref[...]


def _mlp_chunk(c, counts, embb, w1b, b1r, w2b, b2r, w3r, b3r, prev_out):
    blk0 = c * (_CHUNK // _BLK)
    return pl.pallas_call(
        _mlp_body,
        grid=(_CHUNK // _BLK,),
        in_specs=[
            pl.BlockSpec((_BLK, _F), lambda i: (i, 0)),
            pl.BlockSpec((_F, _E), lambda i: (0, 0)),
            pl.BlockSpec((_E, _H), lambda i: (0, 0)),
            pl.BlockSpec((1, _H), lambda i: (0, 0)),
            pl.BlockSpec((_H, _E), lambda i: (0, 0)),
            pl.BlockSpec((1, _E), lambda i: (0, 0)),
            pl.BlockSpec((1, _E), lambda i: (0, 0)),
            pl.BlockSpec((1, 1), lambda i: (0, 0)),
            pl.BlockSpec((_BLK, 1), lambda i: (blk0 + i, 0)),
        ],
        out_specs=pl.BlockSpec((_BLK, 1), lambda i: (blk0 + i, 0)),
        out_shape=jax.ShapeDtypeStruct(prev_out.shape, jnp.float32),
        input_output_aliases={8: 0},
    )(counts, embb, w1b, b1r, w2b, b2r, w3r, b3r, prev_out)


@jax.jit
def kernel(features_indices, emb, W1, b1, W2, b2, W3, b3):
    n = features_indices.shape[0]
    idx = features_indices.astype(jnp.int32)
    embb = emb.astype(jnp.bfloat16)
    w1b = W1.astype(jnp.bfloat16)
    w2b = W2.astype(jnp.bfloat16)
    b1r = b1.reshape(1, _H)
    b2r = b2.reshape(1, _E)
    w3r = W3.reshape(1, _E)
    b3r = b3.reshape(1, 1)
    out = jnp.zeros((n, 1), jnp.float32)
    for c in range(n // _CHUNK):
        counts = _sc_hist(c * _CHUNK)(idx)
        out = _mlp_chunk(c, counts, embb, w1b, b1r, w2b, b2r, w3r, b3r, out)
    return out


# speedup vs baseline: 1.0304x; 1.0304x over previous
import dataclasses
import functools

import jax
import jax.numpy as jnp
from jax import lax
from jax.experimental import pallas as pl
from jax.experimental.pallas import tpu as pltpu
from jax.experimental.pallas import tpu_sc as plsc

_F = 768
_E = 256
_H = 512
_BAG = 32
_CHUNK = 4096
_NW = 32
_SPW = _CHUNK // _NW
_GRP = 32
_NGRP = _SPW // _GRP
_BLK = 1024


def _sc_hist_body(chunk_row0, idx_hbm, cnt_hbm, idx_v, cnt_a, cnt_b,
                  sem_a, sem_b):
    wid = lax.axis_index("s") * 2 + lax.axis_index("c")
    row0 = wid * _SPW
    pltpu.sync_copy(idx_hbm.at[pl.ds(chunk_row0 + row0, _SPW)], idx_v)

    def _dense_zero(cnt):
        @pl.loop(0, _GRP)
        def _zero_r(r):
            @pl.loop(0, _F, step=64)
            def _zero_c(i):
                z = jnp.zeros((16,), jnp.float32)
                for k in range(4):
                    cnt[r, pl.ds(i + 16 * k, 16)] = z

    ones = jnp.full((16,), 1.0, jnp.float32)
    zeros = jnp.zeros((16,), jnp.float32)
    bufs = [(cnt_a, sem_a), (cnt_b, sem_b)]
    copies = [None, None]

    _dense_zero(cnt_a)
    for g in range(_NGRP):
        cnt, sem = bufs[g % 2]
        if copies[g % 2] is not None:
            copies[g % 2].wait()
            pg = g - 2

            def _rezero(t, pg=pg, cnt=cnt):
                rows = jnp.full((16,), t, jnp.int32)
                a = idx_v[pg * _GRP + t, pl.ds(0, 16)]
                b = idx_v[pg * _GRP + t, pl.ds(16, 16)]
                plsc.store_scatter(cnt, [rows, a], zeros)
                plsc.store_scatter(cnt, [rows, b], zeros)

            pl.loop(0, _GRP)(_rezero)

        def _hist(t, g=g, cnt=cnt):
            for u in range(2):
                s = 2 * t + u
                rows = jnp.full((16,), s, jnp.int32)
                a = idx_v[g * _GRP + s, pl.ds(0, 16)]
                b = idx_v[g * _GRP + s, pl.ds(16, 16)]
                plsc.addupdate_scatter(cnt, [rows, a], ones)
                plsc.addupdate_scatter(cnt, [rows, b], ones)

        pl.loop(0, _GRP // 2)(_hist)
        copies[g % 2] = pltpu.async_copy(
            cnt, cnt_hbm.at[pl.ds(row0 + g * _GRP, _GRP)], sem)
        if g == 0:
            _dense_zero(cnt_b)

    copies[(_NGRP - 2) % 2].wait()
    copies[(_NGRP - 1) % 2].wait()


@functools.lru_cache(maxsize=None)
def _sc_hist(chunk_row0):
    cp = pltpu.CompilerParams()
    if "needs_layout_passes" in pltpu.CompilerParams.__dataclass_fields__:
        cp = dataclasses.replace(cp, needs_layout_passes=False)
    return pl.kernel(
        functools.partial(_sc_hist_body, chunk_row0),
        compiler_params=cp,
        out_type=jax.ShapeDtypeStruct((_CHUNK, _F), jnp.float32),
        mesh=plsc.VectorSubcoreMesh(core_axis_name="c", subcore_axis_name="s"),
        scratch_types=[
            pltpu.VMEM((_SPW, _BAG), jnp.int32),
            pltpu.VMEM((_GRP, _F), jnp.float32),
            pltpu.VMEM((_GRP, _F), jnp.float32),
            pltpu.SemaphoreType.DMA,
            pltpu.SemaphoreType.DMA,
        ],
    )


def _mlp_body(cnt_ref, emb_ref, w1_ref, b1_ref, w2_ref, b2_ref, w3_ref,
              b3_ref, *prev_and_out):
    out_ref = prev_and_out[-1]
    c = cnt_ref[...].astype(jnp.bfloat16)
    x = jnp.dot(c, emb_ref[...], preferred_element_type=jnp.float32)
    h1 = jnp.dot(x.astype(jnp.bfloat16), w1_ref[...],
                 preferred_element_type=jnp.float32) + b1_ref[...]
    h1 = jnp.maximum(h1, 0.0).astype(jnp.bfloat16)
    h2 = jnp.dot(h1, w2_ref[...], preferred_element_type=jnp.float32) \
        + b2_ref[...]
    h2 = jnp.maximum(h2, 0.0)
    out_ref[...] = jnp.sum(h2 * w3_ref[...], axis=1, keepdims=True) \
        + b3_ref[...]


def _mlp_chunk(c, n, counts, embb, w1b, b1r, w2b, b2r, w3r, b3r, prev_out):
    blk0 = c * (_CHUNK // _BLK)
    in_specs = [
        pl.BlockSpec((_BLK, _F), lambda i: (i, 0)),
        pl.BlockSpec((_F, _E), lambda i: (0, 0)),
        pl.BlockSpec((_E, _H), lambda i: (0, 0)),
        pl.BlockSpec((1, _H), lambda i: (0, 0)),
        pl.BlockSpec((_H, _E), lambda i: (0, 0)),
        pl.BlockSpec((1, _E), lambda i: (0, 0)),
        pl.BlockSpec((1, _E), lambda i: (0, 0)),
        pl.BlockSpec((1, 1), lambda i: (0, 0)),
    ]
    args = [counts, embb, w1b, b1r, w2b, b2r, w3r, b3r]
    aliases = {}
    if prev_out is not None:
        in_specs.append(pl.BlockSpec(memory_space=pltpu.MemorySpace.HBM))
        args.append(prev_out)
        aliases = {8: 0}
    return pl.pallas_call(
        _mlp_body,
        grid=(_CHUNK // _BLK,),
        in_specs=in_specs,
        out_specs=pl.BlockSpec((_BLK, 1), lambda i: (blk0 + i, 0)),
        out_shape=jax.ShapeDtypeStruct((n, 1), jnp.float32),
        input_output_aliases=aliases,
    )(*args)


@jax.jit
def kernel(features_indices, emb, W1, b1, W2, b2, W3, b3):
    n = features_indices.shape[0]
    idx = features_indices.astype(jnp.int32)
    embb = emb.astype(jnp.bfloat16)
    w1b = W1.astype(jnp.bfloat16)
    w2b = W2.astype(jnp.bfloat16)
    b1r = b1.reshape(1, _H)
    b2r = b2.reshape(1, _E)
    w3r = W3.reshape(1, _E)
    b3r = b3.reshape(1, 1)
    out = None
    for c in range(n // _CHUNK):
        counts = _sc_hist(c * _CHUNK)(idx)
        out = _mlp_chunk(c, n, counts, embb, w1b, b1r, w2b, b2r, w3r, b3r,
                         out)
    return out
